# baseline (speedup 1.0000x reference)
import functools

import jax
import jax.numpy as jnp
from jax import lax
from jax.experimental import pallas as pl
from jax.experimental.pallas import tpu as pltpu
from jax.experimental.pallas import tpu_sc as plsc

_W = 1024
_H = 1024
_F = 16
_B = 262144
_NC = 2
_NS = 16
_NW = _NC * _NS
_PPW = _B // _NW
_CHUNK = 128
_NCHUNK = _PPW // _CHUNK
_G = _CHUNK // 16
_YB = 8


@functools.partial(
    pl.pallas_call,
    out_shape=jax.ShapeDtypeStruct((_H * _W * _F,), jnp.float32),
    grid=(_H // _YB,),
    in_specs=[pl.BlockSpec((_F, _YB, _W), lambda y: (0, y, 0))],
    out_specs=pl.BlockSpec((_YB * _W * _F,), lambda y: (y,)),
    compiler_params=pltpu.CompilerParams(
        dimension_semantics=("parallel",)),
)
def _to_texel_major(src, dst):
  x8 = src[...]
  m = jnp.transpose(x8, (1, 0, 2)).reshape(_YB * _F, _W)
  for c in range(_W // 128):
    t = m[:, c * 128:(c + 1) * 128].T
    dst[pl.ds(c * 128 * _YB * _F, 128 * _YB * _F)] = t.reshape(-1)


@functools.partial(
    pl.pallas_call,
    out_shape=jax.ShapeDtypeStruct((_F, _B), jnp.float32),
    grid=(_B * _F // 131072,),
    in_specs=[pl.BlockSpec((131072,), lambda i: (i,))],
    out_specs=pl.BlockSpec((_F, 8192), lambda i: (0, i)),
    compiler_params=pltpu.CompilerParams(
        dimension_semantics=("parallel",)),
)
def _unshuffle(src, dst):
  for r in range(8):
    t = src[pl.ds(r * 16384, 16384)].reshape(128, 128).T
    for g in range(8):
      dst[:, r * 1024 + g * 128:r * 1024 + (g + 1) * 128] = (
          t[16 * g:16 * (g + 1), :])


@functools.partial(
    pl.kernel,
    out_type=jax.ShapeDtypeStruct((_B * _F // 128, 128), jnp.float32),
    mesh=plsc.VectorSubcoreMesh(core_axis_name="c", subcore_axis_name="s"),
    compiler_params=pltpu.CompilerParams(use_tc_tiling_on_sc=False),
    scratch_types=[
        pltpu.VMEM((_PPW,), jnp.float32),
        pltpu.VMEM((_PPW,), jnp.float32),
        pltpu.VMEM((_CHUNK,), jnp.int32),
        pltpu.VMEM((_CHUNK,), jnp.int32),
        pltpu.VMEM((_CHUNK,), jnp.int32),
        pltpu.VMEM((_CHUNK,), jnp.int32),
        pltpu.VMEM((_CHUNK,), jnp.float32),
        pltpu.VMEM((_CHUNK,), jnp.float32),
        pltpu.VMEM((_CHUNK, _F), jnp.float32),
        pltpu.VMEM((_CHUNK, _F), jnp.float32),
        pltpu.VMEM((_CHUNK, _F), jnp.float32),
        pltpu.VMEM((_CHUNK, _F), jnp.float32),
        pltpu.VMEM((_CHUNK, _F), jnp.float32),
        pltpu.SemaphoreType.DMA,
        pltpu.SemaphoreType.DMA,
        pltpu.SemaphoreType.DMA,
        pltpu.SemaphoreType.DMA,
    ],
)
def _sample(xs_hbm, ys_hbm, tab_hbm, out_hbm,
            xs_v, ys_v, i00_v, i01_v, i10_v, i11_v, fx_v, fy_v,
            g00, g01, g10, g11, out_v, sem0, sem1, sem2, sem3):
  wid = lax.axis_index("s") * _NC + lax.axis_index("c")
  base = wid * _PPW
  pltpu.sync_copy(xs_hbm.at[pl.ds(base, _PPW)], xs_v)
  pltpu.sync_copy(ys_hbm.at[pl.ds(base, _PPW)], ys_v)

  def chunk_body(c, carry):
    off = c * _CHUNK

    def idx_body(g, carry2):
      o = off + g * 16
      u = xs_v[pl.ds(o, 16)]
      v = ys_v[pl.ds(o, 16)]
      x = ((u * 2.0 - 1.0) + 1.0) * 0.5 * float(_W - 1)
      y = ((v * 2.0 - 1.0) + 1.0) * 0.5 * float(_H - 1)
      xi = x.astype(jnp.int32)
      yi = y.astype(jnp.int32)
      s = g * 16
      fx_v[pl.ds(s, 16)] = x - xi.astype(jnp.float32)
      fy_v[pl.ds(s, 16)] = y - yi.astype(jnp.float32)
      x1 = xi + 1
      y1 = yi + 1
      ux0 = ((xi >> 7) << 10) + ((xi & 127) << 3)
      ux1 = ((x1 >> 7) << 10) + ((x1 & 127) << 3)
      uy0 = ((yi >> 3) << 13) + (yi & 7)
      uy1 = ((y1 >> 3) << 13) + (y1 & 7)
      i00_v[pl.ds(s, 16)] = uy0 + ux0
      i01_v[pl.ds(s, 16)] = uy0 + ux1
      i10_v[pl.ds(s, 16)] = uy1 + ux0
      i11_v[pl.ds(s, 16)] = uy1 + ux1
      return carry2

    lax.fori_loop(0, _G, idx_body, 0)

    cp0 = pltpu.async_copy(tab_hbm.at[i00_v], g00, sem0)
    cp1 = pltpu.async_copy(tab_hbm.at[i01_v], g01, sem1)
    cp2 = pltpu.async_copy(tab_hbm.at[i10_v], g10, sem2)
    cp3 = pltpu.async_copy(tab_hbm.at[i11_v], g11, sem3)
    cp0.wait()
    cp1.wait()
    cp2.wait()
    cp3.wait()

    def blend_body(g, carry2):
      s = g * 16
      fxg = fx_v[pl.ds(s, 16)]
      fyg = fy_v[pl.ds(s, 16)]
      for j in range(16):
        i = s + j
        a00 = g00[i, :]
        a01 = g01[i, :]
        a10 = g10[i, :]
        a11 = g11[i, :]
        fx = jnp.full((16,), fxg[j], jnp.float32)
        fy = jnp.full((16,), fyg[j], jnp.float32)
        top = a00 + fx * (a01 - a00)
        bot = a10 + fx * (a11 - a10)
        out_v[i, :] = top + fy * (bot - top)
      return carry2

    lax.fori_loop(0, _G, blend_body, 0)
    kk = (wid * 8 + (c >> 3)) * 128
    gg = (c & 7) * 16
    pltpu.sync_copy(out_v, out_hbm.at[pl.ds(kk, _CHUNK), pl.ds(gg, _F)])
    return carry

  lax.fori_loop(0, _NCHUNK, chunk_body, 0)


def kernel(uv_, params):
  flat_table = _to_texel_major(params[0])
  table = flat_table.reshape(_H * _W, _F)
  xs = uv_[:, 0]
  ys = uv_[:, 1]
  out2 = _sample(xs, ys, table)
  out_fmajor = _unshuffle(out2.reshape(_B * _F))
  return out_fmajor.T


# speedup vs baseline: 1.1316x; 1.1316x over previous
import functools

import jax
import jax.numpy as jnp
from jax import lax
from jax.experimental import pallas as pl
from jax.experimental.pallas import tpu as pltpu
from jax.experimental.pallas import tpu_sc as plsc

_W = 1024
_H = 1024
_F = 16
_B = 262144
_NC = 2
_NS = 16
_NW = _NC * _NS
_PPW = _B // _NW
_CHUNK = 128
_NCHUNK = _PPW // _CHUNK
_G = _CHUNK // 16
_YB = 16


@functools.partial(
    pl.pallas_call,
    out_shape=jax.ShapeDtypeStruct((_H * _W * _F,), jnp.float32),
    grid=(_H // _YB,),
    in_specs=[pl.BlockSpec((_F, _YB, _W), lambda y: (0, y, 0))],
    out_specs=pl.BlockSpec((_YB * _W * _F,), lambda y: (y,)),
    compiler_params=pltpu.CompilerParams(
        dimension_semantics=("parallel",)),
)
def _to_texel_major(src, dst):
  x8 = src[...]
  m = jnp.transpose(x8, (1, 0, 2)).reshape(_YB * _F, _W)
  for c in range(_W // 128):
    t = m[:, c * 128:(c + 1) * 128].T
    dst[pl.ds(c * 128 * _YB * _F, 128 * _YB * _F)] = t.reshape(-1)


@functools.partial(
    pl.pallas_call,
    out_shape=jax.ShapeDtypeStruct((_F, _B), jnp.float32),
    grid=(_B * _F // 131072,),
    in_specs=[pl.BlockSpec((131072,), lambda i: (i,))],
    out_specs=pl.BlockSpec((_F, 8192), lambda i: (0, i)),
    compiler_params=pltpu.CompilerParams(
        dimension_semantics=("parallel",)),
)
def _unshuffle(src, dst):
  for r in range(8):
    t = src[pl.ds(r * 16384, 16384)].reshape(128, 128).T
    for g in range(8):
      dst[:, r * 1024 + g * 128:r * 1024 + (g + 1) * 128] = (
          t[16 * g:16 * (g + 1), :])


@functools.partial(
    pl.kernel,
    out_type=jax.ShapeDtypeStruct((_B * _F // 128, 128), jnp.float32),
    mesh=plsc.VectorSubcoreMesh(core_axis_name="c", subcore_axis_name="s"),
    compiler_params=pltpu.CompilerParams(use_tc_tiling_on_sc=False),
    scratch_types=[
        pltpu.VMEM((_PPW,), jnp.float32),
        pltpu.VMEM((_PPW,), jnp.float32),
        pltpu.VMEM((_CHUNK,), jnp.int32),
        pltpu.VMEM((_CHUNK,), jnp.int32),
        pltpu.VMEM((_CHUNK,), jnp.int32),
        pltpu.VMEM((_CHUNK,), jnp.int32),
        pltpu.VMEM((_CHUNK,), jnp.float32),
        pltpu.VMEM((_CHUNK,), jnp.float32),
        pltpu.VMEM((_CHUNK, _F), jnp.float32),
        pltpu.VMEM((_CHUNK, _F), jnp.float32),
        pltpu.VMEM((_CHUNK, _F), jnp.float32),
        pltpu.VMEM((_CHUNK, _F), jnp.float32),
        pltpu.VMEM((_CHUNK, _F), jnp.float32),
        pltpu.SemaphoreType.DMA,
        pltpu.SemaphoreType.DMA,
        pltpu.SemaphoreType.DMA,
        pltpu.SemaphoreType.DMA,
    ],
)
def _sample(xs_hbm, ys_hbm, tab_hbm, out_hbm,
            xs_v, ys_v, i00_v, i01_v, i10_v, i11_v, fx_v, fy_v,
            g00, g01, g10, g11, out_v, sem0, sem1, sem2, sem3):
  wid = lax.axis_index("s") * _NC + lax.axis_index("c")
  base = wid * _PPW
  pltpu.sync_copy(xs_hbm.at[pl.ds(base, _PPW)], xs_v)
  pltpu.sync_copy(ys_hbm.at[pl.ds(base, _PPW)], ys_v)

  def chunk_body(c, carry):
    off = c * _CHUNK

    def idx_body(g, carry2):
      o = off + g * 16
      u = xs_v[pl.ds(o, 16)]
      v = ys_v[pl.ds(o, 16)]
      x = ((u * 2.0 - 1.0) + 1.0) * 0.5 * float(_W - 1)
      y = ((v * 2.0 - 1.0) + 1.0) * 0.5 * float(_H - 1)
      xi = x.astype(jnp.int32)
      yi = y.astype(jnp.int32)
      s = g * 16
      fx_v[pl.ds(s, 16)] = x - xi.astype(jnp.float32)
      fy_v[pl.ds(s, 16)] = y - yi.astype(jnp.float32)
      x1 = xi + 1
      y1 = yi + 1
      ux0 = ((xi >> 7) << 11) + ((xi & 127) << 4)
      ux1 = ((x1 >> 7) << 11) + ((x1 & 127) << 4)
      uy0 = ((yi >> 4) << 14) + (yi & 15)
      uy1 = ((y1 >> 4) << 14) + (y1 & 15)
      i00_v[pl.ds(s, 16)] = uy0 + ux0
      i01_v[pl.ds(s, 16)] = uy0 + ux1
      i10_v[pl.ds(s, 16)] = uy1 + ux0
      i11_v[pl.ds(s, 16)] = uy1 + ux1
      return carry2

    lax.fori_loop(0, _G, idx_body, 0)

    cp0 = pltpu.async_copy(tab_hbm.at[i00_v], g00, sem0)
    cp1 = pltpu.async_copy(tab_hbm.at[i01_v], g01, sem1)
    cp2 = pltpu.async_copy(tab_hbm.at[i10_v], g10, sem2)
    cp3 = pltpu.async_copy(tab_hbm.at[i11_v], g11, sem3)
    cp0.wait()
    cp1.wait()
    cp2.wait()
    cp3.wait()

    def blend_body(g, carry2):
      s = g * 16
      fxg = fx_v[pl.ds(s, 16)]
      fyg = fy_v[pl.ds(s, 16)]
      for j in range(16):
        i = s + j
        a00 = g00[i, :]
        a01 = g01[i, :]
        a10 = g10[i, :]
        a11 = g11[i, :]
        fx = jnp.full((16,), fxg[j], jnp.float32)
        fy = jnp.full((16,), fyg[j], jnp.float32)
        top = a00 + fx * (a01 - a00)
        bot = a10 + fx * (a11 - a10)
        out_v[i, :] = top + fy * (bot - top)
      return carry2

    lax.fori_loop(0, _G, blend_body, 0)
    kk = (wid * 8 + (c >> 3)) * 128
    gg = (c & 7) * 16
    pltpu.sync_copy(out_v, out_hbm.at[pl.ds(kk, _CHUNK), pl.ds(gg, _F)])
    return carry

  lax.fori_loop(0, _NCHUNK, chunk_body, 0)


def kernel(uv_, params):
  flat_table = _to_texel_major(params[0])
  table = flat_table.reshape(_H * _W, _F)
  xs = uv_[:, 0]
  ys = uv_[:, 1]
  out2 = _sample(xs, ys, table)
  out_fmajor = _unshuffle(out2.reshape(_B * _F))
  return out_fmajor.T


# speedup vs baseline: 1.2272x; 1.0845x over previous
import functools

import jax
import jax.numpy as jnp
from jax import lax
from jax.experimental import pallas as pl
from jax.experimental.pallas import tpu as pltpu
from jax.experimental.pallas import tpu_sc as plsc

_W = 1024
_H = 1024
_F = 16
_B = 262144
_NC = 2
_NS = 16
_NW = _NC * _NS
_PPW = _B // _NW
_CHUNK = 128
_NCHUNK = _PPW // _CHUNK
_G = _CHUNK // 16
_YB = 32


@functools.partial(
    pl.pallas_call,
    out_shape=jax.ShapeDtypeStruct((_H * _W * _F,), jnp.float32),
    grid=(_H // _YB,),
    in_specs=[pl.BlockSpec((_F, _YB, _W), lambda y: (0, y, 0))],
    out_specs=pl.BlockSpec((_YB * _W * _F,), lambda y: (y,)),
    compiler_params=pltpu.CompilerParams(
        dimension_semantics=("parallel",)),
)
def _to_texel_major(src, dst):
  x8 = src[...]
  m = jnp.transpose(x8, (1, 0, 2)).reshape(_YB * _F, _W)
  for c in range(_W // 128):
    t = m[:, c * 128:(c + 1) * 128].T
    dst[pl.ds(c * 128 * _YB * _F, 128 * _YB * _F)] = t.reshape(-1)


@functools.partial(
    pl.pallas_call,
    out_shape=jax.ShapeDtypeStruct((_F, _B), jnp.float32),
    grid=(_B * _F // 131072,),
    in_specs=[pl.BlockSpec((131072,), lambda i: (i,))],
    out_specs=pl.BlockSpec((_F, 8192), lambda i: (0, i)),
    compiler_params=pltpu.CompilerParams(
        dimension_semantics=("parallel",)),
)
def _unshuffle(src, dst):
  for r in range(8):
    t = src[pl.ds(r * 16384, 16384)].reshape(128, 128).T
    for g in range(8):
      dst[:, r * 1024 + g * 128:r * 1024 + (g + 1) * 128] = (
          t[16 * g:16 * (g + 1), :])


@functools.partial(
    pl.kernel,
    out_type=jax.ShapeDtypeStruct((_B * _F // 128, 128), jnp.float32),
    mesh=plsc.VectorSubcoreMesh(core_axis_name="c", subcore_axis_name="s"),
    compiler_params=pltpu.CompilerParams(use_tc_tiling_on_sc=False),
    scratch_types=[
        pltpu.VMEM((_PPW,), jnp.float32),
        pltpu.VMEM((_PPW,), jnp.float32),
        pltpu.VMEM((_CHUNK,), jnp.int32),
        pltpu.VMEM((_CHUNK,), jnp.int32),
        pltpu.VMEM((_CHUNK,), jnp.int32),
        pltpu.VMEM((_CHUNK,), jnp.int32),
        pltpu.VMEM((_CHUNK,), jnp.float32),
        pltpu.VMEM((_CHUNK,), jnp.float32),
        pltpu.VMEM((_CHUNK, _F), jnp.float32),
        pltpu.VMEM((_CHUNK, _F), jnp.float32),
        pltpu.VMEM((_CHUNK, _F), jnp.float32),
        pltpu.VMEM((_CHUNK, _F), jnp.float32),
        pltpu.VMEM((_CHUNK, _F), jnp.float32),
        pltpu.SemaphoreType.DMA,
        pltpu.SemaphoreType.DMA,
        pltpu.SemaphoreType.DMA,
        pltpu.SemaphoreType.DMA,
    ],
)
def _sample(xs_hbm, ys_hbm, tab_hbm, out_hbm,
            xs_v, ys_v, i00_v, i01_v, i10_v, i11_v, fx_v, fy_v,
            g00, g01, g10, g11, out_v, sem0, sem1, sem2, sem3):
  wid = lax.axis_index("s") * _NC + lax.axis_index("c")
  base = wid * _PPW
  pltpu.sync_copy(xs_hbm.at[pl.ds(base, _PPW)], xs_v)
  pltpu.sync_copy(ys_hbm.at[pl.ds(base, _PPW)], ys_v)

  def chunk_body(c, carry):
    off = c * _CHUNK

    def idx_body(g, carry2):
      o = off + g * 16
      u = xs_v[pl.ds(o, 16)]
      v = ys_v[pl.ds(o, 16)]
      x = ((u * 2.0 - 1.0) + 1.0) * 0.5 * float(_W - 1)
      y = ((v * 2.0 - 1.0) + 1.0) * 0.5 * float(_H - 1)
      xi = x.astype(jnp.int32)
      yi = y.astype(jnp.int32)
      s = g * 16
      fx_v[pl.ds(s, 16)] = x - xi.astype(jnp.float32)
      fy_v[pl.ds(s, 16)] = y - yi.astype(jnp.float32)
      x1 = xi + 1
      y1 = yi + 1
      ux0 = ((xi >> 7) << 12) + ((xi & 127) << 5)
      ux1 = ((x1 >> 7) << 12) + ((x1 & 127) << 5)
      uy0 = ((yi >> 5) << 15) + (yi & 31)
      uy1 = ((y1 >> 5) << 15) + (y1 & 31)
      i00_v[pl.ds(s, 16)] = uy0 + ux0
      i01_v[pl.ds(s, 16)] = uy0 + ux1
      i10_v[pl.ds(s, 16)] = uy1 + ux0
      i11_v[pl.ds(s, 16)] = uy1 + ux1
      return carry2

    lax.fori_loop(0, _G, idx_body, 0)

    cp0 = pltpu.async_copy(tab_hbm.at[i00_v], g00, sem0)
    cp1 = pltpu.async_copy(tab_hbm.at[i01_v], g01, sem1)
    cp2 = pltpu.async_copy(tab_hbm.at[i10_v], g10, sem2)
    cp3 = pltpu.async_copy(tab_hbm.at[i11_v], g11, sem3)
    cp0.wait()
    cp1.wait()
    cp2.wait()
    cp3.wait()

    def blend_body(g, carry2):
      s = g * 16
      fxg = fx_v[pl.ds(s, 16)]
      fyg = fy_v[pl.ds(s, 16)]
      for j in range(16):
        i = s + j
        a00 = g00[i, :]
        a01 = g01[i, :]
        a10 = g10[i, :]
        a11 = g11[i, :]
        fx = jnp.full((16,), fxg[j], jnp.float32)
        fy = jnp.full((16,), fyg[j], jnp.float32)
        top = a00 + fx * (a01 - a00)
        bot = a10 + fx * (a11 - a10)
        out_v[i, :] = top + fy * (bot - top)
      return carry2

    lax.fori_loop(0, _G, blend_body, 0)
    kk = (wid * 8 + (c >> 3)) * 128
    gg = (c & 7) * 16
    pltpu.sync_copy(out_v, out_hbm.at[pl.ds(kk, _CHUNK), pl.ds(gg, _F)])
    return carry

  lax.fori_loop(0, _NCHUNK, chunk_body, 0)


def kernel(uv_, params):
  flat_table = _to_texel_major(params[0])
  table = flat_table.reshape(_H * _W, _F)
  xs = uv_[:, 0]
  ys = uv_[:, 1]
  out2 = _sample(xs, ys, table)
  out_fmajor = _unshuffle(out2.reshape(_B * _F))
  return out_fmajor.T


# speedup vs baseline: 1.2636x; 1.0297x over previous
import functools

import jax
import jax.numpy as jnp
from jax import lax
from jax.experimental import pallas as pl
from jax.experimental.pallas import tpu as pltpu
from jax.experimental.pallas import tpu_sc as plsc

_W = 1024
_H = 1024
_F = 16
_B = 262144
_NC = 2
_NS = 16
_NW = _NC * _NS
_PPW = _B // _NW
_CHUNK = 128
_NCHUNK = _PPW // _CHUNK
_G = _CHUNK // 16
_YB = 64


@functools.partial(
    pl.pallas_call,
    out_shape=jax.ShapeDtypeStruct((_H * _W * _F,), jnp.float32),
    grid=(_H // _YB,),
    in_specs=[pl.BlockSpec((_F, _YB, _W), lambda y: (0, y, 0))],
    out_specs=pl.BlockSpec((_YB * _W * _F,), lambda y: (y,)),
    compiler_params=pltpu.CompilerParams(
        dimension_semantics=("parallel",)),
)
def _to_texel_major(src, dst):
  x8 = src[...]
  m = jnp.transpose(x8, (1, 0, 2)).reshape(_YB * _F, _W)
  for c in range(_W // 128):
    t = m[:, c * 128:(c + 1) * 128].T
    dst[pl.ds(c * 128 * _YB * _F, 128 * _YB * _F)] = t.reshape(-1)


@functools.partial(
    pl.pallas_call,
    out_shape=jax.ShapeDtypeStruct((_F, _B), jnp.float32),
    grid=(_B * _F // 131072,),
    in_specs=[pl.BlockSpec((131072,), lambda i: (i,))],
    out_specs=pl.BlockSpec((_F, 8192), lambda i: (0, i)),
    compiler_params=pltpu.CompilerParams(
        dimension_semantics=("parallel",)),
)
def _unshuffle(src, dst):
  for r in range(8):
    t = src[pl.ds(r * 16384, 16384)].reshape(128, 128).T
    for g in range(8):
      dst[:, r * 1024 + g * 128:r * 1024 + (g + 1) * 128] = (
          t[16 * g:16 * (g + 1), :])


@functools.partial(
    pl.kernel,
    out_type=jax.ShapeDtypeStruct((_B * _F // 128, 128), jnp.float32),
    mesh=plsc.VectorSubcoreMesh(core_axis_name="c", subcore_axis_name="s"),
    compiler_params=pltpu.CompilerParams(use_tc_tiling_on_sc=False),
    scratch_types=[
        pltpu.VMEM((_PPW,), jnp.float32),
        pltpu.VMEM((_PPW,), jnp.float32),
        pltpu.VMEM((_CHUNK,), jnp.int32),
        pltpu.VMEM((_CHUNK,), jnp.int32),
        pltpu.VMEM((_CHUNK,), jnp.int32),
        pltpu.VMEM((_CHUNK,), jnp.int32),
        pltpu.VMEM((_CHUNK,), jnp.float32),
        pltpu.VMEM((_CHUNK,), jnp.float32),
        pltpu.VMEM((_CHUNK, _F), jnp.float32),
        pltpu.VMEM((_CHUNK, _F), jnp.float32),
        pltpu.VMEM((_CHUNK, _F), jnp.float32),
        pltpu.VMEM((_CHUNK, _F), jnp.float32),
        pltpu.VMEM((_CHUNK, _F), jnp.float32),
        pltpu.SemaphoreType.DMA,
        pltpu.SemaphoreType.DMA,
        pltpu.SemaphoreType.DMA,
        pltpu.SemaphoreType.DMA,
    ],
)
def _sample(xs_hbm, ys_hbm, tab_hbm, out_hbm,
            xs_v, ys_v, i00_v, i01_v, i10_v, i11_v, fx_v, fy_v,
            g00, g01, g10, g11, out_v, sem0, sem1, sem2, sem3):
  wid = lax.axis_index("s") * _NC + lax.axis_index("c")
  base = wid * _PPW
  pltpu.sync_copy(xs_hbm.at[pl.ds(base, _PPW)], xs_v)
  pltpu.sync_copy(ys_hbm.at[pl.ds(base, _PPW)], ys_v)

  def chunk_body(c, carry):
    off = c * _CHUNK

    def idx_body(g, carry2):
      o = off + g * 16
      u = xs_v[pl.ds(o, 16)]
      v = ys_v[pl.ds(o, 16)]
      x = ((u * 2.0 - 1.0) + 1.0) * 0.5 * float(_W - 1)
      y = ((v * 2.0 - 1.0) + 1.0) * 0.5 * float(_H - 1)
      xi = x.astype(jnp.int32)
      yi = y.astype(jnp.int32)
      s = g * 16
      fx_v[pl.ds(s, 16)] = x - xi.astype(jnp.float32)
      fy_v[pl.ds(s, 16)] = y - yi.astype(jnp.float32)
      x1 = xi + 1
      y1 = yi + 1
      ux0 = ((xi >> 7) << 13) + ((xi & 127) << 6)
      ux1 = ((x1 >> 7) << 13) + ((x1 & 127) << 6)
      uy0 = ((yi >> 6) << 16) + (yi & 63)
      uy1 = ((y1 >> 6) << 16) + (y1 & 63)
      i00_v[pl.ds(s, 16)] = uy0 + ux0
      i01_v[pl.ds(s, 16)] = uy0 + ux1
      i10_v[pl.ds(s, 16)] = uy1 + ux0
      i11_v[pl.ds(s, 16)] = uy1 + ux1
      return carry2

    lax.fori_loop(0, _G, idx_body, 0)

    cp0 = pltpu.async_copy(tab_hbm.at[i00_v], g00, sem0)
    cp1 = pltpu.async_copy(tab_hbm.at[i01_v], g01, sem1)
    cp2 = pltpu.async_copy(tab_hbm.at[i10_v], g10, sem2)
    cp3 = pltpu.async_copy(tab_hbm.at[i11_v], g11, sem3)
    cp0.wait()
    cp1.wait()
    cp2.wait()
    cp3.wait()

    def blend_body(g, carry2):
      s = g * 16
      fxg = fx_v[pl.ds(s, 16)]
      fyg = fy_v[pl.ds(s, 16)]
      for j in range(16):
        i = s + j
        a00 = g00[i, :]
        a01 = g01[i, :]
        a10 = g10[i, :]
        a11 = g11[i, :]
        fx = jnp.full((16,), fxg[j], jnp.float32)
        fy = jnp.full((16,), fyg[j], jnp.float32)
        top = a00 + fx * (a01 - a00)
        bot = a10 + fx * (a11 - a10)
        out_v[i, :] = top + fy * (bot - top)
      return carry2

    lax.fori_loop(0, _G, blend_body, 0)
    kk = (wid * 8 + (c >> 3)) * 128
    gg = (c & 7) * 16
    pltpu.sync_copy(out_v, out_hbm.at[pl.ds(kk, _CHUNK), pl.ds(gg, _F)])
    return carry

  lax.fori_loop(0, _NCHUNK, chunk_body, 0)


def kernel(uv_, params):
  flat_table = _to_texel_major(params[0])
  table = flat_table.reshape(_H * _W, _F)
  xs = uv_[:, 0]
  ys = uv_[:, 1]
  out2 = _sample(xs, ys, table)
  out_fmajor = _unshuffle(out2.reshape(_B * _F))
  return out_fmajor.T


# speedup vs baseline: 1.3375x; 1.0585x over previous
import functools

import jax
import jax.numpy as jnp
from jax import lax
from jax.experimental import pallas as pl
from jax.experimental.pallas import tpu as pltpu
from jax.experimental.pallas import tpu_sc as plsc

_W = 1024
_H = 1024
_F = 16
_B = 262144
_NC = 2
_NS = 16
_NW = _NC * _NS
_PPW = _B // _NW
_CHUNK = 128
_NCHUNK = _PPW // _CHUNK
_G = _CHUNK // 16
_YB = 64


@functools.partial(
    pl.pallas_call,
    out_shape=jax.ShapeDtypeStruct((_H * _W * _F,), jnp.float32),
    grid=(_H // _YB,),
    in_specs=[pl.BlockSpec((_F, _YB, _W), lambda y: (0, y, 0))],
    out_specs=pl.BlockSpec((_YB * _W * _F,), lambda y: (y,)),
    compiler_params=pltpu.CompilerParams(
        dimension_semantics=("parallel",)),
)
def _to_texel_major(src, dst):
  x8 = src[...]
  m = jnp.transpose(x8, (1, 0, 2)).reshape(_YB * _F, _W)
  for c in range(_W // 128):
    t = m[:, c * 128:(c + 1) * 128].T
    dst[pl.ds(c * 128 * _YB * _F, 128 * _YB * _F)] = t.reshape(-1)


@functools.partial(
    pl.pallas_call,
    out_shape=jax.ShapeDtypeStruct((_F, _B), jnp.float32),
    grid=(_B * _F // 524288,),
    in_specs=[pl.BlockSpec((524288,), lambda i: (i,))],
    out_specs=pl.BlockSpec((_F, 32768), lambda i: (0, i)),
    compiler_params=pltpu.CompilerParams(
        dimension_semantics=("parallel",)),
)
def _unshuffle(src, dst):
  for r in range(32):
    t = src[pl.ds(r * 16384, 16384)].reshape(128, 128).T
    for g in range(8):
      dst[:, r * 1024 + g * 128:r * 1024 + (g + 1) * 128] = (
          t[16 * g:16 * (g + 1), :])


@functools.partial(
    pl.kernel,
    out_type=jax.ShapeDtypeStruct((_B * _F // 128, 128), jnp.float32),
    mesh=plsc.VectorSubcoreMesh(core_axis_name="c", subcore_axis_name="s"),
    compiler_params=pltpu.CompilerParams(use_tc_tiling_on_sc=False),
    scratch_types=[
        pltpu.VMEM((_PPW,), jnp.float32),
        pltpu.VMEM((_PPW,), jnp.float32),
        pltpu.VMEM((_CHUNK,), jnp.int32),
        pltpu.VMEM((_CHUNK,), jnp.int32),
        pltpu.VMEM((_CHUNK,), jnp.int32),
        pltpu.VMEM((_CHUNK,), jnp.int32),
        pltpu.VMEM((_CHUNK,), jnp.float32),
        pltpu.VMEM((_CHUNK,), jnp.float32),
        pltpu.VMEM((_CHUNK, _F), jnp.float32),
        pltpu.VMEM((_CHUNK, _F), jnp.float32),
        pltpu.VMEM((_CHUNK, _F), jnp.float32),
        pltpu.VMEM((_CHUNK, _F), jnp.float32),
        pltpu.VMEM((_CHUNK, _F), jnp.float32),
        pltpu.SemaphoreType.DMA,
        pltpu.SemaphoreType.DMA,
        pltpu.SemaphoreType.DMA,
        pltpu.SemaphoreType.DMA,
    ],
)
def _sample(xs_hbm, ys_hbm, tab_hbm, out_hbm,
            xs_v, ys_v, i00_v, i01_v, i10_v, i11_v, fx_v, fy_v,
            g00, g01, g10, g11, out_v, sem0, sem1, sem2, sem3):
  wid = lax.axis_index("s") * _NC + lax.axis_index("c")
  base = wid * _PPW
  pltpu.sync_copy(xs_hbm.at[pl.ds(base, _PPW)], xs_v)
  pltpu.sync_copy(ys_hbm.at[pl.ds(base, _PPW)], ys_v)

  def chunk_body(c, carry):
    off = c * _CHUNK

    def idx_body(g, carry2):
      o = off + g * 16
      u = xs_v[pl.ds(o, 16)]
      v = ys_v[pl.ds(o, 16)]
      x = ((u * 2.0 - 1.0) + 1.0) * 0.5 * float(_W - 1)
      y = ((v * 2.0 - 1.0) + 1.0) * 0.5 * float(_H - 1)
      xi = x.astype(jnp.int32)
      yi = y.astype(jnp.int32)
      s = g * 16
      fx_v[pl.ds(s, 16)] = x - xi.astype(jnp.float32)
      fy_v[pl.ds(s, 16)] = y - yi.astype(jnp.float32)
      x1 = xi + 1
      y1 = yi + 1
      ux0 = ((xi >> 7) << 13) + ((xi & 127) << 6)
      ux1 = ((x1 >> 7) << 13) + ((x1 & 127) << 6)
      uy0 = ((yi >> 6) << 16) + (yi & 63)
      uy1 = ((y1 >> 6) << 16) + (y1 & 63)
      i00_v[pl.ds(s, 16)] = uy0 + ux0
      i01_v[pl.ds(s, 16)] = uy0 + ux1
      i10_v[pl.ds(s, 16)] = uy1 + ux0
      i11_v[pl.ds(s, 16)] = uy1 + ux1
      return carry2

    lax.fori_loop(0, _G, idx_body, 0)

    cp0 = pltpu.async_copy(tab_hbm.at[i00_v], g00, sem0)
    cp1 = pltpu.async_copy(tab_hbm.at[i01_v], g01, sem1)
    cp2 = pltpu.async_copy(tab_hbm.at[i10_v], g10, sem2)
    cp3 = pltpu.async_copy(tab_hbm.at[i11_v], g11, sem3)
    cp0.wait()
    cp1.wait()
    cp2.wait()
    cp3.wait()

    def blend_body(g, carry2):
      s = g * 16
      fxg = fx_v[pl.ds(s, 16)]
      fyg = fy_v[pl.ds(s, 16)]
      for j in range(16):
        i = s + j
        a00 = g00[i, :]
        a01 = g01[i, :]
        a10 = g10[i, :]
        a11 = g11[i, :]
        fx = jnp.full((16,), fxg[j], jnp.float32)
        fy = jnp.full((16,), fyg[j], jnp.float32)
        top = a00 + fx * (a01 - a00)
        bot = a10 + fx * (a11 - a10)
        out_v[i, :] = top + fy * (bot - top)
      return carry2

    lax.fori_loop(0, _G, blend_body, 0)
    kk = (wid * 8 + (c >> 3)) * 128
    gg = (c & 7) * 16
    pltpu.sync_copy(out_v, out_hbm.at[pl.ds(kk, _CHUNK), pl.ds(gg, _F)])
    return carry

  lax.fori_loop(0, _NCHUNK, chunk_body, 0)


def kernel(uv_, params):
  flat_table = _to_texel_major(params[0])
  table = flat_table.reshape(_H * _W, _F)
  xs = uv_[:, 0]
  ys = uv_[:, 1]
  out2 = _sample(xs, ys, table)
  out_fmajor = _unshuffle(out2.reshape(_B * _F))
  return out_fmajor.T


# speedup vs baseline: 1.3479x; 1.0078x over previous
import functools

import jax
import jax.numpy as jnp
from jax import lax
from jax.experimental import pallas as pl
from jax.experimental.pallas import tpu as pltpu
from jax.experimental.pallas import tpu_sc as plsc

_W = 1024
_H = 1024
_F = 16
_B = 262144
_NC = 2
_NS = 16
_NW = _NC * _NS
_PPW = _B // _NW
_CHUNK = 128
_NCHUNK = _PPW // _CHUNK
_G = _CHUNK // 16
_YB = 128


@functools.partial(
    pl.pallas_call,
    out_shape=jax.ShapeDtypeStruct((_H * _W * _F,), jnp.float32),
    grid=(_H // _YB,),
    in_specs=[pl.BlockSpec((_F, _YB, _W), lambda y: (0, y, 0))],
    out_specs=pl.BlockSpec((_YB * _W * _F,), lambda y: (y,)),
    compiler_params=pltpu.CompilerParams(
        dimension_semantics=("parallel",)),
)
def _to_texel_major(src, dst):
  x8 = src[...]
  m = jnp.transpose(x8, (1, 0, 2)).reshape(_YB * _F, _W)
  for c in range(_W // 128):
    t = m[:, c * 128:(c + 1) * 128].T
    dst[pl.ds(c * 128 * _YB * _F, 128 * _YB * _F)] = t.reshape(-1)


@functools.partial(
    pl.pallas_call,
    out_shape=jax.ShapeDtypeStruct((_F, _B), jnp.float32),
    grid=(_B * _F // 524288,),
    in_specs=[pl.BlockSpec((524288,), lambda i: (i,))],
    out_specs=pl.BlockSpec((_F, 32768), lambda i: (0, i)),
    compiler_params=pltpu.CompilerParams(
        dimension_semantics=("parallel",)),
)
def _unshuffle(src, dst):
  for r in range(32):
    t = src[pl.ds(r * 16384, 16384)].reshape(128, 128).T
    for g in range(8):
      dst[:, r * 1024 + g * 128:r * 1024 + (g + 1) * 128] = (
          t[16 * g:16 * (g + 1), :])


@functools.partial(
    pl.kernel,
    out_type=jax.ShapeDtypeStruct((_B * _F // 128, 128), jnp.float32),
    mesh=plsc.VectorSubcoreMesh(core_axis_name="c", subcore_axis_name="s"),
    compiler_params=pltpu.CompilerParams(use_tc_tiling_on_sc=False),
    scratch_types=[
        pltpu.VMEM((_PPW,), jnp.float32),
        pltpu.VMEM((_PPW,), jnp.float32),
        pltpu.VMEM((_CHUNK,), jnp.int32),
        pltpu.VMEM((_CHUNK,), jnp.int32),
        pltpu.VMEM((_CHUNK,), jnp.int32),
        pltpu.VMEM((_CHUNK,), jnp.int32),
        pltpu.VMEM((_CHUNK,), jnp.float32),
        pltpu.VMEM((_CHUNK,), jnp.float32),
        pltpu.VMEM((_CHUNK, _F), jnp.float32),
        pltpu.VMEM((_CHUNK, _F), jnp.float32),
        pltpu.VMEM((_CHUNK, _F), jnp.float32),
        pltpu.VMEM((_CHUNK, _F), jnp.float32),
        pltpu.VMEM((_CHUNK, _F), jnp.float32),
        pltpu.SemaphoreType.DMA,
        pltpu.SemaphoreType.DMA,
        pltpu.SemaphoreType.DMA,
        pltpu.SemaphoreType.DMA,
    ],
)
def _sample(xs_hbm, ys_hbm, tab_hbm, out_hbm,
            xs_v, ys_v, i00_v, i01_v, i10_v, i11_v, fx_v, fy_v,
            g00, g01, g10, g11, out_v, sem0, sem1, sem2, sem3):
  wid = lax.axis_index("s") * _NC + lax.axis_index("c")
  base = wid * _PPW
  pltpu.sync_copy(xs_hbm.at[pl.ds(base, _PPW)], xs_v)
  pltpu.sync_copy(ys_hbm.at[pl.ds(base, _PPW)], ys_v)

  def chunk_body(c, carry):
    off = c * _CHUNK

    def idx_body(g, carry2):
      o = off + g * 16
      u = xs_v[pl.ds(o, 16)]
      v = ys_v[pl.ds(o, 16)]
      x = ((u * 2.0 - 1.0) + 1.0) * 0.5 * float(_W - 1)
      y = ((v * 2.0 - 1.0) + 1.0) * 0.5 * float(_H - 1)
      xi = x.astype(jnp.int32)
      yi = y.astype(jnp.int32)
      s = g * 16
      fx_v[pl.ds(s, 16)] = x - xi.astype(jnp.float32)
      fy_v[pl.ds(s, 16)] = y - yi.astype(jnp.float32)
      x1 = xi + 1
      y1 = yi + 1
      ux0 = ((xi >> 7) << 14) + ((xi & 127) << 7)
      ux1 = ((x1 >> 7) << 14) + ((x1 & 127) << 7)
      uy0 = ((yi >> 7) << 17) + (yi & 127)
      uy1 = ((y1 >> 7) << 17) + (y1 & 127)
      i00_v[pl.ds(s, 16)] = uy0 + ux0
      i01_v[pl.ds(s, 16)] = uy0 + ux1
      i10_v[pl.ds(s, 16)] = uy1 + ux0
      i11_v[pl.ds(s, 16)] = uy1 + ux1
      return carry2

    lax.fori_loop(0, _G, idx_body, 0)

    cp0 = pltpu.async_copy(tab_hbm.at[i00_v], g00, sem0)
    cp1 = pltpu.async_copy(tab_hbm.at[i01_v], g01, sem1)
    cp2 = pltpu.async_copy(tab_hbm.at[i10_v], g10, sem2)
    cp3 = pltpu.async_copy(tab_hbm.at[i11_v], g11, sem3)
    cp0.wait()
    cp1.wait()
    cp2.wait()
    cp3.wait()

    def blend_body(g, carry2):
      s = g * 16
      fxg = fx_v[pl.ds(s, 16)]
      fyg = fy_v[pl.ds(s, 16)]
      for j in range(16):
        i = s + j
        a00 = g00[i, :]
        a01 = g01[i, :]
        a10 = g10[i, :]
        a11 = g11[i, :]
        fx = jnp.full((16,), fxg[j], jnp.float32)
        fy = jnp.full((16,), fyg[j], jnp.float32)
        top = a00 + fx * (a01 - a00)
        bot = a10 + fx * (a11 - a10)
        out_v[i, :] = top + fy * (bot - top)
      return carry2

    lax.fori_loop(0, _G, blend_body, 0)
    kk = (wid * 8 + (c >> 3)) * 128
    gg = (c & 7) * 16
    pltpu.sync_copy(out_v, out_hbm.at[pl.ds(kk, _CHUNK), pl.ds(gg, _F)])
    return carry

  lax.fori_loop(0, _NCHUNK, chunk_body, 0)


def kernel(uv_, params):
  flat_table = _to_texel_major(params[0])
  table = flat_table.reshape(_H * _W, _F)
  xs = uv_[:, 0]
  ys = uv_[:, 1]
  out2 = _sample(xs, ys, table)
  out_fmajor = _unshuffle(out2.reshape(_B * _F))
  return out_fmajor.T
